# Initial kernel scaffold; baseline (speedup 1.0000x reference)
import jax
import jax.numpy as jnp
from jax.experimental import pallas as pl


def kernel(a, b, add_in_width0, add_in_width1):
    raise NotImplementedError("write your pallas kernel here")


# speedup vs baseline: 1.5189x; 1.5189x over previous
import jax
import jax.numpy as jnp
import numpy as np
from jax.experimental import pallas as pl
from jax.experimental.pallas import tpu as pltpu


def kernel(a, b, add_in_width0, add_in_width1):
    T1, B, W1, Hh1, Hw1 = a.shape
    W2, H2 = b.shape
    H1 = Hh1 * Hw1
    buffer_size = 32 * 1024
    max_h1 = 256
    max_w2 = 256

    cond_elif = (H1 <= max_h1) or (W2 <= max_w2)
    ceil_w2 = float(np.ceil(W2 / 256))
    load_first_aw0_coef = float(H1 * W1 * T1) / 32.0
    load_aw1_coef = float(H2 * W2) / 32.0
    load_elif_aw0_coef = float(H1 * W1 * T1) * ceil_w2 / 32.0
    compute_lat = float((1 + H1 + 4) * np.ceil(W1 / 16) * ceil_w2) * T1
    lif_lat = float(H1) * ceil_w2 * T1
    store_lat = float(H1 * W2) / 32.0 * T1

    aw = jnp.stack(
        [jnp.asarray(add_in_width0, jnp.float32),
         jnp.asarray(add_in_width1, jnp.float32)]
    )

    def body(aw_ref, out_ref):
        aw0 = aw_ref[0]
        aw1 = aw_ref[1]
        max_w1_h2 = buffer_size / (256.0 * aw0 / 8.0 + 16.0 * aw1 / 8.0)
        cond_first = jnp.logical_or(
            jnp.logical_and(H1 <= max_h1, W1 <= max_w1_h2),
            jnp.logical_and(H2 <= max_w1_h2, W2 <= max_w2),
        )
        active = jnp.logical_or(cond_first, cond_elif)
        load_first = load_first_aw0_coef * aw0 + load_aw1_coef * aw1
        load_elif = load_elif_aw0_coef * aw0 + load_aw1_coef * aw1
        load_latency = jnp.where(
            cond_first, load_first,
            jnp.where(cond_elif, load_elif, 0.0),
        )
        compute_latency = jnp.where(active, compute_lat, 0.0)
        lif_latency = jnp.where(active, lif_lat, 0.0)
        store_latency = jnp.where(active, store_lat, 0.0)
        latency_a = (load_latency + compute_latency
                     + lif_latency + store_latency)
        out_ref[0] = latency_a * B
        out_ref[1] = load_latency * B
        out_ref[2] = compute_latency * B
        out_ref[3] = lif_latency * B
        out_ref[4] = store_latency * B

    out = pl.pallas_call(
        body,
        out_shape=jax.ShapeDtypeStruct((5,), jnp.float32),
        in_specs=[pl.BlockSpec(memory_space=pltpu.SMEM)],
        out_specs=pl.BlockSpec(memory_space=pltpu.SMEM),
    )(aw)

    return (out[0], out[1], out[2], out[3], out[4])


# speedup vs baseline: 2.9134x; 1.9181x over previous
import jax
import jax.numpy as jnp
import numpy as np
from jax.experimental import pallas as pl
from jax.experimental.pallas import tpu as pltpu


def kernel(a, b, add_in_width0, add_in_width1):
    T1, B, W1, Hh1, Hw1 = a.shape
    W2, H2 = b.shape
    H1 = Hh1 * Hw1
    buffer_size = 32 * 1024
    max_h1 = 256
    max_w2 = 256

    cond_elif = (H1 <= max_h1) or (W2 <= max_w2)
    ceil_w2 = float(np.ceil(W2 / 256))
    load_first_aw0_coef = float(H1 * W1 * T1) / 32.0
    load_aw1_coef = float(H2 * W2) / 32.0
    load_elif_aw0_coef = float(H1 * W1 * T1) * ceil_w2 / 32.0
    compute_lat = float((1 + H1 + 4) * np.ceil(W1 / 16) * ceil_w2) * T1
    lif_lat = float(H1) * ceil_w2 * T1
    store_lat = float(H1 * W2) / 32.0 * T1

    def body(aw0_ref, aw1_ref, *out_refs):
        aw0 = aw0_ref[...]
        aw1 = aw1_ref[...]
        max_w1_h2 = buffer_size / (256.0 * aw0 / 8.0 + 16.0 * aw1 / 8.0)
        cond_first = jnp.logical_or(
            jnp.logical_and(H1 <= max_h1, W1 <= max_w1_h2),
            jnp.logical_and(H2 <= max_w1_h2, W2 <= max_w2),
        )
        active = jnp.logical_or(cond_first, cond_elif)
        load_first = load_first_aw0_coef * aw0 + load_aw1_coef * aw1
        load_elif = load_elif_aw0_coef * aw0 + load_aw1_coef * aw1
        load_latency = jnp.where(
            cond_first, load_first,
            jnp.where(cond_elif, load_elif, 0.0),
        )
        compute_latency = jnp.where(active, compute_lat, 0.0)
        lif_latency = jnp.where(active, lif_lat, 0.0)
        store_latency = jnp.where(active, store_lat, 0.0)
        latency_a = (load_latency + compute_latency
                     + lif_latency + store_latency)
        out_refs[0][...] = latency_a * B
        out_refs[1][...] = load_latency * B
        out_refs[2][...] = compute_latency * B
        out_refs[3][...] = lif_latency * B
        out_refs[4][...] = store_latency * B

    return pl.pallas_call(
        body,
        out_shape=tuple(
            jax.ShapeDtypeStruct((), jnp.float32) for _ in range(5)),
        in_specs=[pl.BlockSpec(memory_space=pltpu.SMEM)] * 2,
        out_specs=tuple(pl.BlockSpec(memory_space=pltpu.SMEM)
                        for _ in range(5)),
    )(jnp.asarray(add_in_width0, jnp.float32),
      jnp.asarray(add_in_width1, jnp.float32))


# speedup vs baseline: 2.9857x; 1.0248x over previous
import jax
import jax.numpy as jnp
import numpy as np
from jax.experimental import pallas as pl
from jax.experimental.pallas import tpu as pltpu


def kernel(a, b, add_in_width0, add_in_width1):
    T1, B, W1, Hh1, Hw1 = a.shape
    W2, H2 = b.shape
    H1 = Hh1 * Hw1
    buffer_size = 32 * 1024
    max_h1 = 256
    max_w2 = 256

    cond_elif = (H1 <= max_h1) or (W2 <= max_w2)
    ceil_w2 = float(np.ceil(W2 / 256))
    load_first_aw0_coef = float(H1 * W1 * T1) / 32.0
    load_aw1_coef = float(H2 * W2) / 32.0
    load_elif_aw0_coef = float(H1 * W1 * T1) * ceil_w2 / 32.0
    compute_lat = float((1 + H1 + 4) * np.ceil(W1 / 16) * ceil_w2) * T1
    lif_lat = float(H1) * ceil_w2 * T1
    store_lat = float(H1 * W2) / 32.0 * T1

    def body(aw0_ref, aw1_ref, *out_refs):
        aw0 = aw0_ref[...].astype(jnp.float32)
        aw1 = aw1_ref[...].astype(jnp.float32)
        max_w1_h2 = buffer_size / (256.0 * aw0 / 8.0 + 16.0 * aw1 / 8.0)
        cond_first = jnp.logical_or(
            jnp.logical_and(H1 <= max_h1, W1 <= max_w1_h2),
            jnp.logical_and(H2 <= max_w1_h2, W2 <= max_w2),
        )
        active = jnp.logical_or(cond_first, cond_elif)
        load_first = load_first_aw0_coef * aw0 + load_aw1_coef * aw1
        load_elif = load_elif_aw0_coef * aw0 + load_aw1_coef * aw1
        load_latency = jnp.where(
            cond_first, load_first,
            jnp.where(cond_elif, load_elif, 0.0),
        )
        compute_latency = jnp.where(active, compute_lat, 0.0)
        lif_latency = jnp.where(active, lif_lat, 0.0)
        store_latency = jnp.where(active, store_lat, 0.0)
        latency_a = (load_latency + compute_latency
                     + lif_latency + store_latency)
        out_refs[0][...] = latency_a * B
        out_refs[1][...] = load_latency * B
        out_refs[2][...] = compute_latency * B
        out_refs[3][...] = lif_latency * B
        out_refs[4][...] = store_latency * B

    return pl.pallas_call(
        body,
        out_shape=tuple(
            jax.ShapeDtypeStruct((), jnp.float32) for _ in range(5)),
        in_specs=[pl.BlockSpec(memory_space=pltpu.SMEM)] * 2,
        out_specs=tuple(pl.BlockSpec(memory_space=pltpu.SMEM)
                        for _ in range(5)),
    )(add_in_width0, add_in_width1)
